# initial kernel scaffold (unmeasured)
import jax
import jax.numpy as jnp
from jax import lax
from jax.experimental import pallas as pl
from jax.experimental.pallas import tpu as pltpu

N_DEV = 16
B_BLK = 512
D_MODEL = 256
N_ROWS = N_DEV * B_BLK
CHUNK = 2048

_MESH = pl.DeviceIdType.MESH


def kernel(x, Win0, Wout0, Win1, Wout1, Win2, Wout2):
    def body(x_ref, win0_ref, wout0_ref, win1_ref, wout1_ref, win2_ref,
             wout2_ref, out_ref, xfull_ref, pbf_ref, rsbuf_ref,
             send_sems, rs_sems, ag_sems):
        me = lax.axis_index("i")

        bar = pltpu.get_barrier_semaphore()
        for d in range(1, N_DEV):
            peer = lax.rem(me + d, N_DEV)
            pl.semaphore_signal(bar, inc=1, device_id=(peer,),
                                device_id_type=_MESH)
        pl.semaphore_wait(bar, N_DEV - 1)

        def all_exchange_blocks(buf_ref):
            descs = []
            for d in range(1, N_DEV):
                peer = lax.rem(me + d, N_DEV)
                c = pltpu.make_async_remote_copy(
                    src_ref=buf_ref.at[pl.ds(me * B_BLK, B_BLK)],
                    dst_ref=buf_ref.at[pl.ds(me * B_BLK, B_BLK)],
                    send_sem=send_sems.at[d - 1],
                    recv_sem=ag_sems.at[me],
                    device_id=(peer,),
                    device_id_type=_MESH,
                )
                c.start()
                descs.append(c)
            for d in range(1, N_DEV):
                src = lax.rem(me + d, N_DEV)
                pltpu.make_async_remote_copy(
                    src_ref=buf_ref.at[pl.ds(src * B_BLK, B_BLK)],
                    dst_ref=buf_ref.at[pl.ds(src * B_BLK, B_BLK)],
                    send_sem=send_sems.at[d - 1],
                    recv_sem=ag_sems.at[src],
                    device_id=(src,),
                    device_id_type=_MESH,
                ).wait_recv()
            for c in descs:
                c.wait_send()

        def layer_compute(win_ref, wout_ref):
            wi = win_ref[...].astype(jnp.bfloat16)
            wo = wout_ref[...].astype(jnp.bfloat16)
            for r in range(0, N_ROWS, CHUNK):
                xc = xfull_ref[pl.ds(r, CHUNK)]
                h = jnp.dot(xc, wi, preferred_element_type=jnp.float32)
                h = jnp.maximum(h, 0.0).astype(jnp.bfloat16)
                p = jnp.dot(h, wo, preferred_element_type=jnp.float32)
                pbf_ref[pl.ds(r, CHUNK)] = p.astype(jnp.bfloat16)

        def scatter_reduce_partials():
            descs = []
            for d in range(1, N_DEV):
                peer = lax.rem(me + d, N_DEV)
                c = pltpu.make_async_remote_copy(
                    src_ref=pbf_ref.at[pl.ds(peer * B_BLK, B_BLK)],
                    dst_ref=rsbuf_ref.at[me],
                    send_sem=send_sems.at[d - 1],
                    recv_sem=rs_sems.at[me],
                    device_id=(peer,),
                    device_id_type=_MESH,
                )
                c.start()
                descs.append(c)
            rsbuf_ref[me, :, :] = pbf_ref[pl.ds(me * B_BLK, B_BLK)]
            for d in range(1, N_DEV):
                src = lax.rem(me + d, N_DEV)
                pltpu.make_async_remote_copy(
                    src_ref=pbf_ref.at[pl.ds(me * B_BLK, B_BLK)],
                    dst_ref=rsbuf_ref.at[src],
                    send_sem=send_sems.at[d - 1],
                    recv_sem=rs_sems.at[src],
                    device_id=(src,),
                    device_id_type=_MESH,
                ).wait_recv()
            for c in descs:
                c.wait_send()
            return jnp.sum(rsbuf_ref[...].astype(jnp.float32), axis=0)

        xfull_ref[pl.ds(me * B_BLK, B_BLK)] = x_ref[...].astype(jnp.bfloat16)
        all_exchange_blocks(xfull_ref)

        layers = [(win0_ref, wout0_ref), (win1_ref, wout1_ref),
                  (win2_ref, wout2_ref)]
        for k, (win_ref, wout_ref) in enumerate(layers):
            layer_compute(win_ref, wout_ref)
            red = scatter_reduce_partials()
            if k < 2:
                xfull_ref[pl.ds(me * B_BLK, B_BLK)] = red.astype(jnp.bfloat16)
                all_exchange_blocks(xfull_ref)
            else:
                out_ref[pl.ds(me * B_BLK, B_BLK)] = red
                all_exchange_blocks(out_ref)

    out_shape = jax.ShapeDtypeStruct((N_ROWS, D_MODEL), jnp.float32)
    return pl.pallas_call(
        body,
        out_shape=out_shape,
        in_specs=[pl.BlockSpec(memory_space=pltpu.VMEM)] * 7,
        out_specs=pl.BlockSpec(memory_space=pltpu.VMEM),
        scratch_shapes=[
            pltpu.VMEM((N_ROWS, D_MODEL), jnp.bfloat16),
            pltpu.VMEM((N_ROWS, D_MODEL), jnp.bfloat16),
            pltpu.VMEM((N_DEV, B_BLK, D_MODEL), jnp.bfloat16),
            pltpu.SemaphoreType.DMA((N_DEV,)),
            pltpu.SemaphoreType.DMA((N_DEV,)),
            pltpu.SemaphoreType.DMA((N_DEV,)),
        ],
        compiler_params=pltpu.CompilerParams(collective_id=0),
    )(x, Win0, Wout0, Win1, Wout1, Win2, Wout2)


# baseline (device time: 289120 ns/iter reference)
import jax
import jax.numpy as jnp
from jax import lax
from jax.experimental import pallas as pl
from jax.experimental.pallas import tpu as pltpu

N_DEV = 16
B_BLK = 512
D_MODEL = 256
N_ROWS = N_DEV * B_BLK
GRP = 4
SLAB = GRP * B_BLK
CHUNK = 2048

_MESH = pl.DeviceIdType.MESH


def kernel(x, Win0, Wout0, Win1, Wout1, Win2, Wout2):
    def body(x_ref, win0_ref, wout0_ref, win1_ref, wout1_ref, win2_ref,
             wout2_ref, out_ref, xfull_ref, pbf_ref, csum_ref, prbuf_ref,
             zrbuf_ref, send_sems, prs_sems, zrs_sems, zag_sems, pag_sems):
        me = lax.axis_index("i")
        p = lax.rem(me, GRP)
        z = lax.div(me, GRP)

        bar = pltpu.get_barrier_semaphore()
        for d in range(1, GRP):
            plane_peer = GRP * z + lax.rem(p + d, GRP)
            col_peer = GRP * lax.rem(z + d, GRP) + p
            pl.semaphore_signal(bar, inc=1, device_id=(plane_peer,),
                                device_id_type=_MESH)
            pl.semaphore_signal(bar, inc=1, device_id=(col_peer,),
                                device_id_type=_MESH)
        pl.semaphore_wait(bar, 2 * (GRP - 1))

        def z_allgather(buf_ref, blk_rows):
            my_off = p * SLAB + z * blk_rows
            descs = []
            for d in range(1, GRP):
                peer = GRP * lax.rem(z + d, GRP) + p
                c = pltpu.make_async_remote_copy(
                    src_ref=buf_ref.at[pl.ds(my_off, blk_rows)],
                    dst_ref=buf_ref.at[pl.ds(my_off, blk_rows)],
                    send_sem=send_sems.at[d - 1],
                    recv_sem=zag_sems.at[GRP - 1 - d],
                    device_id=(peer,),
                    device_id_type=_MESH,
                )
                c.start()
                descs.append(c)
            for d in range(1, GRP):
                src_z = lax.rem(z + d, GRP)
                off = p * SLAB + src_z * blk_rows
                pltpu.make_async_remote_copy(
                    src_ref=buf_ref.at[pl.ds(off, blk_rows)],
                    dst_ref=buf_ref.at[pl.ds(off, blk_rows)],
                    send_sem=send_sems.at[d - 1],
                    recv_sem=zag_sems.at[d - 1],
                    device_id=(me,),
                    device_id_type=_MESH,
                ).wait_recv()
            for c in descs:
                c.wait_send()

        def plane_allgather(buf_ref):
            descs = []
            for d in range(1, GRP):
                peer = GRP * z + lax.rem(p + d, GRP)
                c = pltpu.make_async_remote_copy(
                    src_ref=buf_ref.at[pl.ds(p * SLAB, SLAB)],
                    dst_ref=buf_ref.at[pl.ds(p * SLAB, SLAB)],
                    send_sem=send_sems.at[d - 1],
                    recv_sem=pag_sems.at[GRP - 1 - d],
                    device_id=(peer,),
                    device_id_type=_MESH,
                )
                c.start()
                descs.append(c)
            for d in range(1, GRP):
                src_p = lax.rem(p + d, GRP)
                pltpu.make_async_remote_copy(
                    src_ref=buf_ref.at[pl.ds(src_p * SLAB, SLAB)],
                    dst_ref=buf_ref.at[pl.ds(src_p * SLAB, SLAB)],
                    send_sem=send_sems.at[d - 1],
                    recv_sem=pag_sems.at[d - 1],
                    device_id=(me,),
                    device_id_type=_MESH,
                ).wait_recv()
            for c in descs:
                c.wait_send()

        def layer_compute(win_ref, wout_ref):
            wi = win_ref[...].astype(jnp.bfloat16)
            wo = wout_ref[...].astype(jnp.bfloat16)
            for r in range(0, N_ROWS, CHUNK):
                xc = xfull_ref[pl.ds(r, CHUNK)]
                h = jnp.dot(xc, wi, preferred_element_type=jnp.float32)
                h = jnp.maximum(h, 0.0).astype(jnp.bfloat16)
                pr = jnp.dot(h, wo, preferred_element_type=jnp.float32)
                pbf_ref[pl.ds(r, CHUNK)] = pr.astype(jnp.bfloat16)

        def reduce_scatter():
            descs = []
            for d in range(1, GRP):
                dst_p = lax.rem(p + d, GRP)
                peer = GRP * z + dst_p
                c = pltpu.make_async_remote_copy(
                    src_ref=pbf_ref.at[pl.ds(dst_p * SLAB, SLAB)],
                    dst_ref=prbuf_ref.at[GRP - 1 - d],
                    send_sem=send_sems.at[d - 1],
                    recv_sem=prs_sems.at[GRP - 1 - d],
                    device_id=(peer,),
                    device_id_type=_MESH,
                )
                c.start()
                descs.append(c)
            for d in range(1, GRP):
                pltpu.make_async_remote_copy(
                    src_ref=prbuf_ref.at[d - 1],
                    dst_ref=prbuf_ref.at[d - 1],
                    send_sem=send_sems.at[d - 1],
                    recv_sem=prs_sems.at[d - 1],
                    device_id=(me,),
                    device_id_type=_MESH,
                ).wait_recv()
            for c in descs:
                c.wait_send()
            cs = (pbf_ref[pl.ds(p * SLAB, SLAB)].astype(jnp.float32)
                  + prbuf_ref[0].astype(jnp.float32)
                  + prbuf_ref[1].astype(jnp.float32)
                  + prbuf_ref[2].astype(jnp.float32))
            csum_ref[...] = cs.astype(jnp.bfloat16)

            descs = []
            for d in range(1, GRP):
                dst_z = lax.rem(z + d, GRP)
                peer = GRP * dst_z + p
                c = pltpu.make_async_remote_copy(
                    src_ref=csum_ref.at[pl.ds(dst_z * B_BLK, B_BLK)],
                    dst_ref=zrbuf_ref.at[GRP - 1 - d],
                    send_sem=send_sems.at[d - 1],
                    recv_sem=zrs_sems.at[GRP - 1 - d],
                    device_id=(peer,),
                    device_id_type=_MESH,
                )
                c.start()
                descs.append(c)
            for d in range(1, GRP):
                pltpu.make_async_remote_copy(
                    src_ref=zrbuf_ref.at[d - 1],
                    dst_ref=zrbuf_ref.at[d - 1],
                    send_sem=send_sems.at[d - 1],
                    recv_sem=zrs_sems.at[d - 1],
                    device_id=(me,),
                    device_id_type=_MESH,
                ).wait_recv()
            for c in descs:
                c.wait_send()
            return (csum_ref[pl.ds(z * B_BLK, B_BLK)].astype(jnp.float32)
                    + zrbuf_ref[0].astype(jnp.float32)
                    + zrbuf_ref[1].astype(jnp.float32)
                    + zrbuf_ref[2].astype(jnp.float32))

        my_off = p * SLAB + z * B_BLK
        xfull_ref[pl.ds(my_off, B_BLK)] = x_ref[...].astype(jnp.bfloat16)
        z_allgather(xfull_ref, B_BLK)
        plane_allgather(xfull_ref)

        layers = [(win0_ref, wout0_ref), (win1_ref, wout1_ref),
                  (win2_ref, wout2_ref)]
        for win_ref, wout_ref in layers:
            layer_compute(win_ref, wout_ref)
            red = reduce_scatter()
            xfull_ref[pl.ds(my_off, B_BLK)] = red.astype(jnp.bfloat16)
            z_allgather(xfull_ref, B_BLK)
            plane_allgather(xfull_ref)

        for b in range(N_DEV):
            bp, bz = b % GRP, b // GRP
            out_ref[pl.ds(b * B_BLK, B_BLK)] = xfull_ref[
                pl.ds(bp * SLAB + bz * B_BLK, B_BLK)].astype(jnp.float32)

    out_shape = jax.ShapeDtypeStruct((N_ROWS, D_MODEL), jnp.float32)
    return pl.pallas_call(
        body,
        out_shape=out_shape,
        in_specs=[pl.BlockSpec(memory_space=pltpu.VMEM)] * 7,
        out_specs=pl.BlockSpec(memory_space=pltpu.VMEM),
        scratch_shapes=[
            pltpu.VMEM((N_ROWS, D_MODEL), jnp.bfloat16),
            pltpu.VMEM((N_ROWS, D_MODEL), jnp.bfloat16),
            pltpu.VMEM((SLAB, D_MODEL), jnp.bfloat16),
            pltpu.VMEM((GRP - 1, SLAB, D_MODEL), jnp.bfloat16),
            pltpu.VMEM((GRP - 1, B_BLK, D_MODEL), jnp.bfloat16),
            pltpu.SemaphoreType.DMA((GRP - 1,)),
            pltpu.SemaphoreType.DMA((GRP - 1,)),
            pltpu.SemaphoreType.DMA((GRP - 1,)),
            pltpu.SemaphoreType.DMA((GRP - 1,)),
            pltpu.SemaphoreType.DMA((GRP - 1,)),
        ],
        compiler_params=pltpu.CompilerParams(collective_id=0),
    )(x, Win0, Wout0, Win1, Wout1, Win2, Wout2)
